# baseline (speedup 1.0000x reference)
import functools

import jax
import jax.numpy as jnp
from jax import lax
from jax.experimental import pallas as pl
from jax.experimental.pallas import tpu as pltpu
from jax.experimental.pallas import tpu_sc as plsc

_MAX_SEQ_LEN = 8192
_EMBED = 1024
_BATCH = 4
_SEQ = 4096

_NC = 2
_ROWS_PER_C = _SEQ // _NC
_CHUNK = 512
_NCHUNK = _ROWS_PER_C // _CHUNK


def _make_sc_kernel():
    mesh = plsc.ScalarSubcoreMesh(axis_name="c", num_cores=_NC)

    @functools.partial(
        pl.kernel,
        mesh=mesh,
        out_type=jax.ShapeDtypeStruct((_BATCH, _SEQ, _EMBED), jnp.float32),
        scratch_types=[
            pltpu.VMEM_SHARED((_CHUNK, _EMBED), jnp.float32),
            pltpu.VMEM_SHARED((_CHUNK, _EMBED), jnp.float32),
            pltpu.SemaphoreType.DMA,
            pltpu.SemaphoreType.DMA,
            pltpu.SemaphoreType.DMA,
            pltpu.SemaphoreType.DMA,
        ],
    )
    def pos_embed_broadcast(table_hbm, out_hbm, buf0, buf1, sr0, sr1, sw0, sw1):
        cid = lax.axis_index("c")
        base = cid * _ROWS_PER_C
        bufs = (buf0, buf1)
        rsems = (sr0, sr1)
        wsems = (sw0, sw1)

        def start_read(c):
            r0 = base + c * _CHUNK
            return pltpu.async_copy(
                table_hbm.at[pl.ds(r0, _CHUNK)], bufs[c % 2], rsems[c % 2])

        def start_writes(c):
            r0 = base + c * _CHUNK
            return [
                pltpu.async_copy(
                    bufs[c % 2], out_hbm.at[b, pl.ds(r0, _CHUNK)], wsems[c % 2])
                for b in range(_BATCH)
            ]

        rh = [None] * _NCHUNK
        wh = [None] * _NCHUNK
        rh[0] = start_read(0)
        for c in range(_NCHUNK):
            if c + 1 < _NCHUNK:
                if c - 1 >= 0:
                    for h in wh[c - 1]:
                        h.wait()
                rh[c + 1] = start_read(c + 1)
            rh[c].wait()
            wh[c] = start_writes(c)
        for c in (_NCHUNK - 2, _NCHUNK - 1):
            for h in wh[c]:
                h.wait()

    return pos_embed_broadcast


_sc_kernel = _make_sc_kernel()


def kernel(x, table):
    del x
    return _sc_kernel(table)


# speedup vs baseline: 1.3125x; 1.3125x over previous
import functools

import jax
import jax.numpy as jnp
from jax import lax
from jax.experimental import pallas as pl
from jax.experimental.pallas import tpu as pltpu
from jax.experimental.pallas import tpu_sc as plsc

_MAX_SEQ_LEN = 8192
_EMBED = 1024
_BATCH = 4
_SEQ = 4096

_NC = 2
_NS = 16
_NW = _NC * _NS
_ROWS_PER_W = _SEQ // _NW
_CHUNK = 64
_NCHUNK = _ROWS_PER_W // _CHUNK


def _make_sc_kernel():
    mesh = plsc.VectorSubcoreMesh(core_axis_name="c", subcore_axis_name="s")

    @functools.partial(
        pl.kernel,
        mesh=mesh,
        out_type=jax.ShapeDtypeStruct((_BATCH, _SEQ, _EMBED), jnp.float32),
        scratch_types=[pltpu.VMEM((_CHUNK, _EMBED), jnp.float32)],
    )
    def pos_embed_broadcast(table_hbm, out_hbm, buf):
        wid = lax.axis_index("s") * _NC + lax.axis_index("c")
        base = wid * _ROWS_PER_W
        for c in range(_NCHUNK):
            r0 = base + c * _CHUNK
            pltpu.sync_copy(table_hbm.at[pl.ds(r0, _CHUNK)], buf)
            for b in range(_BATCH):
                pltpu.sync_copy(buf, out_hbm.at[b, pl.ds(r0, _CHUNK)])

    return pos_embed_broadcast


_sc_kernel = _make_sc_kernel()


def kernel(x, table):
    del x
    return _sc_kernel(table)
